# Initial kernel scaffold; baseline (speedup 1.0000x reference)
import jax
import jax.numpy as jnp
from jax.experimental import pallas as pl


def kernel(x, edge_index, alpha_sp, alpha_ctx, alpha_lat, w_sp, w_ctx, w_lat, W_sp0, W_ctx0, W_lat0, W_self0, W_sp1, W_ctx1, W_lat1, W_self1, b0, b1, dp0, dp1, ln_g0, ln_b0):
    raise NotImplementedError("write your pallas kernel here")


# speedup vs baseline: 9.2802x; 9.2802x over previous
import functools

import jax
import jax.numpy as jnp
from jax import lax
from jax.experimental import pallas as pl
from jax.experimental.pallas import tpu as pltpu
from jax.experimental.pallas import tpu_sc as plsc

N = 10000
E = 320000
D = 128
D3 = 3 * D

NC = 2
NS = 16
NW = NC * NS
EPW = E // NW
K = 80
NCHUNK = EPW // K
NP = 10240
RPT = NP // NS

_SC_MESH = plsc.VectorSubcoreMesh(core_axis_name="c", subcore_axis_name="s")


def _tc_transform_body(x_ref, w3_ref, wself_ref, y_ref, self_ref):
    xb = x_ref[...]
    dn = (((1,), (1,)), ((), ()))
    y_ref[...] = lax.dot_general(xb, w3_ref[...], dn,
                                 preferred_element_type=jnp.float32)
    self_ref[...] = lax.dot_general(xb, wself_ref[...], dn,
                                    preferred_element_type=jnp.float32)


def _tc_transform(x, w3, wself):
    R = 2000
    return pl.pallas_call(
        _tc_transform_body,
        grid=(N // R,),
        in_specs=[
            pl.BlockSpec((R, D), lambda i: (i, 0)),
            pl.BlockSpec((D3, D), lambda i: (0, 0)),
            pl.BlockSpec((D, D), lambda i: (0, 0)),
        ],
        out_specs=[
            pl.BlockSpec((R, D3), lambda i: (i, 0)),
            pl.BlockSpec((R, D), lambda i: (i, 0)),
        ],
        out_shape=[
            jax.ShapeDtypeStruct((N, D3), jnp.float32),
            jax.ShapeDtypeStruct((N, D), jnp.float32),
        ],
    )(x, w3, wself)


def _sc_degree_body(tgt_hbm, asp_hbm, actx_hbm, alat_hbm,
                    wsp_hbm, wctx_hbm, wlat_hbm, z1_hbm,
                    out_hbm,
                    tgt_v, asp_v, actx_v, alat_v, wsp_v, wctx_v, wlat_v,
                    eff_sp_v, eff_ctx_v, eff_lat_v,
                    acc_sp, acc_ctx, acc_lat):
    c = lax.axis_index("c")
    s = lax.axis_index("s")
    wid = s * NC + c
    r0 = s * RPT
    for acc in (acc_sp, acc_ctx, acc_lat):
        pltpu.sync_copy(z1_hbm.at[pl.ds(r0, RPT)], acc.at[pl.ds(r0, RPT)])
    plsc.subcore_barrier()

    base = wid * EPW
    a_refs = (asp_v, actx_v, alat_v)
    w_refs = (wsp_v, wctx_v, wlat_v)
    e_refs = (eff_sp_v, eff_ctx_v, eff_lat_v)
    accs = (acc_sp, acc_ctx, acc_lat)
    a_hbms = (asp_hbm, actx_hbm, alat_hbm)
    w_hbms = (wsp_hbm, wctx_hbm, wlat_hbm)

    def chunk(ci, carry):
        e0 = base + ci * K
        pltpu.sync_copy(tgt_hbm.at[pl.ds(e0, K)], tgt_v)
        for ch in range(3):
            pltpu.sync_copy(a_hbms[ch].at[pl.ds(e0, K)], a_refs[ch])
            pltpu.sync_copy(w_hbms[ch].at[pl.ds(e0, K)], w_refs[ch])
        for ch in range(3):
            for g in range(K // 16):
                sl = pl.ds(g * 16, 16)
                e_refs[ch][sl] = a_refs[ch][sl] * w_refs[ch][sl]
        for ch in range(3):
            pltpu.sync_copy(e_refs[ch], accs[ch].at[tgt_v], add=True)
        return carry

    lax.fori_loop(0, NCHUNK, chunk, 0)
    plsc.subcore_barrier()
    for ch in range(3):
        pltpu.sync_copy(accs[ch].at[pl.ds(r0, RPT)],
                        out_hbm.at[pl.ds(c * (3 * NP) + ch * NP + r0, RPT)])


def _sc_degree(tgt, asp, actx, alat, wsp, wctx, wlat):
    z1 = jnp.zeros((NP,), jnp.float32)
    f = pl.kernel(
        _sc_degree_body,
        out_type=jax.ShapeDtypeStruct((NC * 3 * NP,), jnp.float32),
        mesh=_SC_MESH,
        scratch_types=[
            pltpu.VMEM((K,), jnp.int32),
            pltpu.VMEM((K,), jnp.float32),
            pltpu.VMEM((K,), jnp.float32),
            pltpu.VMEM((K,), jnp.float32),
            pltpu.VMEM((K,), jnp.float32),
            pltpu.VMEM((K,), jnp.float32),
            pltpu.VMEM((K,), jnp.float32),
            pltpu.VMEM((K,), jnp.float32),
            pltpu.VMEM((K,), jnp.float32),
            pltpu.VMEM((K,), jnp.float32),
            pltpu.VMEM_SHARED((NP,), jnp.float32),
            pltpu.VMEM_SHARED((NP,), jnp.float32),
            pltpu.VMEM_SHARED((NP,), jnp.float32),
        ],
    )
    return f(tgt, asp, actx, alat, wsp, wctx, wlat, z1).reshape(NC, 3, NP)


def _tc_dpow_body(deg2_ref, dp0_ref, dp1_ref, d0_ref, d1_ref):
    deg = 1.0 + deg2_ref[0] + deg2_ref[1]
    degc = jnp.maximum(deg, 1e-6)
    d0_ref[...] = jnp.power(degc, dp0_ref[...])
    d1_ref[...] = jnp.power(degc, dp1_ref[...])


def _tc_dpow(deg2, dpb0, dpb1):
    R = 2048
    return pl.pallas_call(
        _tc_dpow_body,
        grid=(NP // R,),
        in_specs=[
            pl.BlockSpec((NC, 3, R), lambda i: (0, 0, i)),
            pl.BlockSpec((3, R), lambda i: (0, i)),
            pl.BlockSpec((3, R), lambda i: (0, i)),
        ],
        out_specs=[
            pl.BlockSpec((3, R), lambda i: (0, i)),
            pl.BlockSpec((3, R), lambda i: (0, i)),
        ],
        out_shape=[
            jax.ShapeDtypeStruct((3, NP), jnp.float32),
            jax.ShapeDtypeStruct((3, NP), jnp.float32),
        ],
    )(deg2, dpb0, dpb1)


def _sc_msgpass_body(y_hbm, src_hbm, tgt_hbm, asp_hbm, actx_hbm, alat_hbm,
                     wsp_hbm, wctx_hbm, wlat_hbm,
                     dsp_hbm, dctx_hbm, dlat_hbm, zd_hbm,
                     out_hbm,
                     src_v, tgt_v,
                     asp_v, actx_v, alat_v, wsp_v, wctx_v, wlat_v,
                     dssp_v, dsctx_v, dslat_v, dtsp_v, dtctx_v, dtlat_v,
                     csp_v, cctx_v, clat_v, rows_v, msg_v, acc_sh, sem):
    c = lax.axis_index("c")
    s = lax.axis_index("s")
    wid = s * NC + c
    r0 = s * RPT
    pltpu.sync_copy(zd_hbm.at[pl.ds(r0, RPT)], acc_sh.at[pl.ds(r0, RPT)])
    plsc.subcore_barrier()

    base = wid * EPW
    a_refs = (asp_v, actx_v, alat_v)
    w_refs = (wsp_v, wctx_v, wlat_v)
    ds_refs = (dssp_v, dsctx_v, dslat_v)
    dt_refs = (dtsp_v, dtctx_v, dtlat_v)
    c_refs = (csp_v, cctx_v, clat_v)
    a_hbms = (asp_hbm, actx_hbm, alat_hbm)
    w_hbms = (wsp_hbm, wctx_hbm, wlat_hbm)
    d_hbms = (dsp_hbm, dctx_hbm, dlat_hbm)

    def chunk(ci, carry):
        e0 = base + ci * K
        pltpu.sync_copy(src_hbm.at[pl.ds(e0, K)], src_v)
        pltpu.sync_copy(tgt_hbm.at[pl.ds(e0, K)], tgt_v)
        for ch in range(3):
            pltpu.sync_copy(a_hbms[ch].at[pl.ds(e0, K)], a_refs[ch])
            pltpu.sync_copy(w_hbms[ch].at[pl.ds(e0, K)], w_refs[ch])
        cps = [pltpu.async_copy(y_hbm.at[src_v], rows_v, sem)]
        for ch in range(3):
            cps.append(pltpu.async_copy(d_hbms[ch].at[src_v], ds_refs[ch], sem))
            cps.append(pltpu.async_copy(d_hbms[ch].at[tgt_v], dt_refs[ch], sem))
        for cp in cps:
            cp.wait()

        for g in range(K // 16):
            sl = pl.ds(g * 16, 16)
            for ch in range(3):
                c_refs[ch][sl] = (a_refs[ch][sl] * w_refs[ch][sl]
                                  * ds_refs[ch][sl] * dt_refs[ch][sl])

        def group(g, carry2):
            j0 = g * 16
            cs = csp_v[pl.ds(j0, 16)]
            cc = cctx_v[pl.ds(j0, 16)]
            cl = clat_v[pl.ds(j0, 16)]
            for jj in range(16):
                c0 = cs[jj]
                c1 = cc[jj]
                c2 = cl[jj]
                j = j0 + jj
                for g8 in range(D // 16):
                    off = g8 * 16
                    v0 = rows_v[j, pl.ds(off, 16)]
                    v1 = rows_v[j, pl.ds(D + off, 16)]
                    v2 = rows_v[j, pl.ds(2 * D + off, 16)]
                    msg_v[j, pl.ds(off, 16)] = c0 * v0 + c1 * v1 + c2 * v2
            return carry2

        lax.fori_loop(0, K // 16, group, 0)
        pltpu.sync_copy(msg_v, acc_sh.at[tgt_v], add=True)
        return carry

    lax.fori_loop(0, NCHUNK, chunk, 0)
    plsc.subcore_barrier()
    pltpu.sync_copy(acc_sh.at[pl.ds(r0, RPT)], out_hbm.at[c, pl.ds(r0, RPT)])


def _sc_msgpass(y, src, tgt, asp, actx, alat, wsp, wctx, wlat, d3np):
    zd = jnp.zeros((NP, D), jnp.float32)
    f = pl.kernel(
        _sc_msgpass_body,
        out_type=jax.ShapeDtypeStruct((NC, NP, D), jnp.float32),
        mesh=_SC_MESH,
        scratch_types=(
            [pltpu.VMEM((K,), jnp.int32)] * 2
            + [pltpu.VMEM((K,), jnp.float32)] * 15
            + [
                pltpu.VMEM((K, D3), jnp.float32),
                pltpu.VMEM((K, D), jnp.float32),
                pltpu.VMEM_SHARED((NP, D), jnp.float32),
                pltpu.SemaphoreType.DMA,
            ]
        ),
    )
    return f(y, src, tgt, asp, actx, alat, wsp, wctx, wlat,
             d3np[0], d3np[1], d3np[2], zd)


def _tc_mid_body(p_ref, self_ref, b0_ref, g_ref, bln_ref, w3_ref, wself_ref,
                 b1_ref, y1_ref, self1_ref):
    conv = p_ref[0] + p_ref[1] + self_ref[...] + b0_ref[...]
    mu = jnp.mean(conv, axis=-1, keepdims=True)
    var = jnp.mean((conv - mu) ** 2, axis=-1, keepdims=True)
    h = (conv - mu) / jnp.sqrt(var + 1e-5) * g_ref[...] + bln_ref[...]
    h = jnp.maximum(h, 0.0)
    dn = (((1,), (1,)), ((), ()))
    y1_ref[...] = lax.dot_general(h, w3_ref[...], dn,
                                  preferred_element_type=jnp.float32)
    self1_ref[...] = lax.dot_general(h, wself_ref[...], dn,
                                     preferred_element_type=jnp.float32) \
        + b1_ref[...]


def _tc_mid(p, self0, b0, g, bln, w3, wself, b1):
    R = 2000
    return pl.pallas_call(
        _tc_mid_body,
        grid=(N // R,),
        in_specs=[
            pl.BlockSpec((NC, R, D), lambda i: (0, i, 0)),
            pl.BlockSpec((R, D), lambda i: (i, 0)),
            pl.BlockSpec((1, D), lambda i: (0, 0)),
            pl.BlockSpec((1, D), lambda i: (0, 0)),
            pl.BlockSpec((1, D), lambda i: (0, 0)),
            pl.BlockSpec((D3, D), lambda i: (0, 0)),
            pl.BlockSpec((D, D), lambda i: (0, 0)),
            pl.BlockSpec((1, D), lambda i: (0, 0)),
        ],
        out_specs=[
            pl.BlockSpec((R, D3), lambda i: (i, 0)),
            pl.BlockSpec((R, D), lambda i: (i, 0)),
        ],
        out_shape=[
            jax.ShapeDtypeStruct((N, D3), jnp.float32),
            jax.ShapeDtypeStruct((N, D), jnp.float32),
        ],
    )(p, self0, b0, g, bln, w3, wself, b1)


def _tc_final_body(p_ref, selfb_ref, out_ref):
    out_ref[...] = p_ref[0] + p_ref[1] + selfb_ref[...]


def _tc_final(p, selfb):
    R = 2000
    return pl.pallas_call(
        _tc_final_body,
        grid=(N // R,),
        in_specs=[
            pl.BlockSpec((NC, R, D), lambda i: (0, i, 0)),
            pl.BlockSpec((R, D), lambda i: (i, 0)),
        ],
        out_specs=pl.BlockSpec((R, D), lambda i: (i, 0)),
        out_shape=jax.ShapeDtypeStruct((N, D), jnp.float32),
    )(p, selfb)


def kernel(x, edge_index, alpha_sp, alpha_ctx, alpha_lat, w_sp, w_ctx, w_lat,
           W_sp0, W_ctx0, W_lat0, W_self0, W_sp1, W_ctx1, W_lat1, W_self1,
           b0, b1, dp0, dp1, ln_g0, ln_b0):
    src = edge_index[0]
    tgt = edge_index[1]
    w3_0 = jnp.concatenate([W_sp0, W_ctx0, W_lat0], axis=0)
    w3_1 = jnp.concatenate([W_sp1, W_ctx1, W_lat1], axis=0)

    y0, self0 = _tc_transform(x, w3_0, W_self0)

    deg2 = _sc_degree(tgt, alpha_sp, alpha_ctx, alpha_lat, w_sp, w_ctx, w_lat)

    dpb0 = jnp.broadcast_to(dp0[:, None], (3, NP))
    dpb1 = jnp.broadcast_to(dp1[:, None], (3, NP))
    d0, d1 = _tc_dpow(deg2, dpb0, dpb1)

    p0 = _sc_msgpass(y0, src, tgt, alpha_sp, alpha_ctx, alpha_lat,
                     w_sp, w_ctx, w_lat, d0)[:, :N]
    y1, self1b = _tc_mid(p0, self0, b0.reshape(1, D), ln_g0.reshape(1, D),
                         ln_b0.reshape(1, D), w3_1, W_self1, b1.reshape(1, D))
    p1 = _sc_msgpass(y1, src, tgt, alpha_sp, alpha_ctx, alpha_lat,
                     w_sp, w_ctx, w_lat, d1)[:, :N]
    return _tc_final(p1, self1b)


# speedup vs baseline: 13.2156x; 1.4241x over previous
import functools

import jax
import jax.numpy as jnp
from jax import lax
from jax.experimental import pallas as pl
from jax.experimental.pallas import tpu as pltpu
from jax.experimental.pallas import tpu_sc as plsc

N = 10000
E = 320000
D = 128
D3 = 3 * D

NC = 2
NS = 16
NW = NC * NS
EPW = E // NW
K = 80
NCHUNK = EPW // K
NP = 10240
RPT = NP // NS

_SC_MESH = plsc.VectorSubcoreMesh(core_axis_name="c", subcore_axis_name="s")


def _tc_transform_body(x_ref, w3_ref, wself_ref, y_ref, self_ref):
    xb = x_ref[...]
    dn = (((1,), (1,)), ((), ()))
    y_ref[...] = lax.dot_general(xb, w3_ref[...], dn,
                                 preferred_element_type=jnp.float32)
    self_ref[...] = lax.dot_general(xb, wself_ref[...], dn,
                                    preferred_element_type=jnp.float32)


def _tc_transform(x, w3, wself):
    R = 2000
    return pl.pallas_call(
        _tc_transform_body,
        grid=(N // R,),
        in_specs=[
            pl.BlockSpec((R, D), lambda i: (i, 0)),
            pl.BlockSpec((D3, D), lambda i: (0, 0)),
            pl.BlockSpec((D, D), lambda i: (0, 0)),
        ],
        out_specs=[
            pl.BlockSpec((R, D3), lambda i: (i, 0)),
            pl.BlockSpec((R, D), lambda i: (i, 0)),
        ],
        out_shape=[
            jax.ShapeDtypeStruct((N, D3), jnp.float32),
            jax.ShapeDtypeStruct((N, D), jnp.float32),
        ],
    )(x, w3, wself)


def _tc_eff_body(a_ref, w_ref, eff_ref):
    eff_ref[...] = a_ref[...] * w_ref[...]


def _tc_eff(a3, w3):
    RB = 64000
    return pl.pallas_call(
        _tc_eff_body,
        grid=(E // RB,),
        in_specs=[
            pl.BlockSpec((3, RB), lambda i: (0, i)),
            pl.BlockSpec((3, RB), lambda i: (0, i)),
        ],
        out_specs=pl.BlockSpec((3, RB), lambda i: (0, i)),
        out_shape=jax.ShapeDtypeStruct((3, E), jnp.float32),
    )(a3, w3)


def _sc_degree_body(tgt_hbm, effpack_hbm, z1_hbm,
                    out_hbm,
                    tgt_v, eff_v, acc_sp, acc_ctx, acc_lat, sem):
    c = lax.axis_index("c")
    s = lax.axis_index("s")
    wid = s * NC + c
    r0 = s * RPT
    for acc in (acc_sp, acc_ctx, acc_lat):
        pltpu.sync_copy(z1_hbm.at[pl.ds(r0, RPT)], acc.at[pl.ds(r0, RPT)])
    plsc.subcore_barrier()

    accs = (acc_sp, acc_ctx, acc_lat)

    def chunk(ci, carry):
        e0 = wid * EPW + ci * K
        p0 = (wid * NCHUNK + ci) * (3 * K)
        cp1 = pltpu.async_copy(tgt_hbm.at[pl.ds(e0, K)], tgt_v, sem)
        cp2 = pltpu.async_copy(effpack_hbm.at[pl.ds(p0, 3 * K)], eff_v, sem)
        cp1.wait()
        cp2.wait()
        cps = [pltpu.async_copy(eff_v.at[pl.ds(ch * K, K)],
                                accs[ch].at[tgt_v], sem, add=True)
               for ch in range(3)]
        for cp in cps:
            cp.wait()
        return carry

    lax.fori_loop(0, NCHUNK, chunk, 0)
    plsc.subcore_barrier()
    for ch in range(3):
        pltpu.sync_copy(accs[ch].at[pl.ds(r0, RPT)],
                        out_hbm.at[pl.ds(c * (3 * NP) + ch * NP + r0, RPT)])


def _sc_degree(tgt, effpack):
    z1 = jnp.zeros((NP,), jnp.float32)
    f = pl.kernel(
        _sc_degree_body,
        out_type=jax.ShapeDtypeStruct((NC * 3 * NP,), jnp.float32),
        mesh=_SC_MESH,
        scratch_types=[
            pltpu.VMEM((K,), jnp.int32),
            pltpu.VMEM((3 * K,), jnp.float32),
            pltpu.VMEM_SHARED((NP,), jnp.float32),
            pltpu.VMEM_SHARED((NP,), jnp.float32),
            pltpu.VMEM_SHARED((NP,), jnp.float32),
            pltpu.SemaphoreType.DMA,
        ],
    )
    return f(tgt, effpack, z1).reshape(NC, 3, NP)


def _tc_dpow_body(deg2_ref, dp0_ref, dp1_ref, d0_ref, d1_ref):
    deg = 1.0 + deg2_ref[0] + deg2_ref[1]
    degc = jnp.maximum(deg, 1e-6)
    d0_ref[...] = jnp.power(degc, dp0_ref[...])
    d1_ref[...] = jnp.power(degc, dp1_ref[...])


def _tc_dpow(deg2, dpb0, dpb1):
    R = 2048
    return pl.pallas_call(
        _tc_dpow_body,
        grid=(NP // R,),
        in_specs=[
            pl.BlockSpec((NC, 3, R), lambda i: (0, 0, i)),
            pl.BlockSpec((3, R), lambda i: (0, i)),
            pl.BlockSpec((3, R), lambda i: (0, i)),
        ],
        out_specs=[
            pl.BlockSpec((3, R), lambda i: (0, i)),
            pl.BlockSpec((3, R), lambda i: (0, i)),
        ],
        out_shape=[
            jax.ShapeDtypeStruct((3, NP), jnp.float32),
            jax.ShapeDtypeStruct((3, NP), jnp.float32),
        ],
    )(deg2, dpb0, dpb1)


def _sc_msgpass_body(y_hbm, src_hbm, tgt_hbm, effpack_hbm,
                     dsp_hbm, dctx_hbm, dlat_hbm, zd_hbm,
                     out_hbm,
                     src_v, tgt_v, eff_v,
                     dssp_v, dsctx_v, dslat_v, dtsp_v, dtctx_v, dtlat_v,
                     csp_v, cctx_v, clat_v, rows_v, msg_v, acc_sh, sem):
    c = lax.axis_index("c")
    s = lax.axis_index("s")
    wid = s * NC + c
    r0 = s * RPT
    pltpu.sync_copy(zd_hbm.at[pl.ds(r0, RPT)], acc_sh.at[pl.ds(r0, RPT)])
    plsc.subcore_barrier()

    ds_refs = (dssp_v, dsctx_v, dslat_v)
    dt_refs = (dtsp_v, dtctx_v, dtlat_v)
    c_refs = (csp_v, cctx_v, clat_v)
    d_hbms = (dsp_hbm, dctx_hbm, dlat_hbm)

    def chunk(ci, carry):
        e0 = wid * EPW + ci * K
        p0 = (wid * NCHUNK + ci) * (3 * K)
        l1 = pltpu.async_copy(src_hbm.at[pl.ds(e0, K)], src_v, sem)
        l2 = pltpu.async_copy(tgt_hbm.at[pl.ds(e0, K)], tgt_v, sem)
        l3 = pltpu.async_copy(effpack_hbm.at[pl.ds(p0, 3 * K)], eff_v, sem)
        l1.wait()
        l2.wait()
        cps = [pltpu.async_copy(y_hbm.at[src_v], rows_v, sem)]
        for ch in range(3):
            cps.append(pltpu.async_copy(d_hbms[ch].at[src_v], ds_refs[ch], sem))
            cps.append(pltpu.async_copy(d_hbms[ch].at[tgt_v], dt_refs[ch], sem))
        l3.wait()
        for cp in cps:
            cp.wait()

        for g in range(K // 16):
            sl = pl.ds(g * 16, 16)
            for ch in range(3):
                c_refs[ch][sl] = (eff_v[pl.ds(ch * K + g * 16, 16)]
                                  * ds_refs[ch][sl] * dt_refs[ch][sl])

        def group(g, carry2):
            j0 = g * 16
            cs = csp_v[pl.ds(j0, 16)]
            cc = cctx_v[pl.ds(j0, 16)]
            cl = clat_v[pl.ds(j0, 16)]
            for jj in range(16):
                c0 = cs[jj]
                c1 = cc[jj]
                c2 = cl[jj]
                j = j0 + jj
                for g8 in range(D // 16):
                    off = g8 * 16
                    v0 = rows_v[j, pl.ds(off, 16)]
                    v1 = rows_v[j, pl.ds(D + off, 16)]
                    v2 = rows_v[j, pl.ds(2 * D + off, 16)]
                    msg_v[j, pl.ds(off, 16)] = c0 * v0 + c1 * v1 + c2 * v2
            return carry2

        lax.fori_loop(0, K // 16, group, 0)
        pltpu.sync_copy(msg_v, acc_sh.at[tgt_v], add=True)
        return carry

    lax.fori_loop(0, NCHUNK, chunk, 0)
    plsc.subcore_barrier()
    pltpu.sync_copy(acc_sh.at[pl.ds(r0, RPT)], out_hbm.at[c, pl.ds(r0, RPT)])


def _sc_msgpass(y, src, tgt, effpack, d3np):
    zd = jnp.zeros((NP, D), jnp.float32)
    f = pl.kernel(
        _sc_msgpass_body,
        out_type=jax.ShapeDtypeStruct((NC, NP, D), jnp.float32),
        mesh=_SC_MESH,
        scratch_types=(
            [pltpu.VMEM((K,), jnp.int32)] * 2
            + [pltpu.VMEM((3 * K,), jnp.float32)]
            + [pltpu.VMEM((K,), jnp.float32)] * 9
            + [
                pltpu.VMEM((K, D3), jnp.float32),
                pltpu.VMEM((K, D), jnp.float32),
                pltpu.VMEM_SHARED((NP, D), jnp.float32),
                pltpu.SemaphoreType.DMA,
            ]
        ),
    )
    return f(y, src, tgt, effpack, d3np[0], d3np[1], d3np[2], zd)


def _tc_mid_body(p_ref, self_ref, b0_ref, g_ref, bln_ref, w3_ref, wself_ref,
                 b1_ref, y1_ref, self1_ref):
    conv = p_ref[0] + p_ref[1] + self_ref[...] + b0_ref[...]
    mu = jnp.mean(conv, axis=-1, keepdims=True)
    var = jnp.mean((conv - mu) ** 2, axis=-1, keepdims=True)
    h = (conv - mu) / jnp.sqrt(var + 1e-5) * g_ref[...] + bln_ref[...]
    h = jnp.maximum(h, 0.0)
    dn = (((1,), (1,)), ((), ()))
    y1_ref[...] = lax.dot_general(h, w3_ref[...], dn,
                                  preferred_element_type=jnp.float32)
    self1_ref[...] = lax.dot_general(h, wself_ref[...], dn,
                                     preferred_element_type=jnp.float32) \
        + b1_ref[...]


def _tc_mid(p, self0, b0, g, bln, w3, wself, b1):
    R = 2000
    return pl.pallas_call(
        _tc_mid_body,
        grid=(N // R,),
        in_specs=[
            pl.BlockSpec((NC, R, D), lambda i: (0, i, 0)),
            pl.BlockSpec((R, D), lambda i: (i, 0)),
            pl.BlockSpec((1, D), lambda i: (0, 0)),
            pl.BlockSpec((1, D), lambda i: (0, 0)),
            pl.BlockSpec((1, D), lambda i: (0, 0)),
            pl.BlockSpec((D3, D), lambda i: (0, 0)),
            pl.BlockSpec((D, D), lambda i: (0, 0)),
            pl.BlockSpec((1, D), lambda i: (0, 0)),
        ],
        out_specs=[
            pl.BlockSpec((R, D3), lambda i: (i, 0)),
            pl.BlockSpec((R, D), lambda i: (i, 0)),
        ],
        out_shape=[
            jax.ShapeDtypeStruct((N, D3), jnp.float32),
            jax.ShapeDtypeStruct((N, D), jnp.float32),
        ],
    )(p, self0, b0, g, bln, w3, wself, b1)


def _tc_final_body(p_ref, selfb_ref, out_ref):
    out_ref[...] = p_ref[0] + p_ref[1] + selfb_ref[...]


def _tc_final(p, selfb):
    R = 2000
    return pl.pallas_call(
        _tc_final_body,
        grid=(N // R,),
        in_specs=[
            pl.BlockSpec((NC, R, D), lambda i: (0, i, 0)),
            pl.BlockSpec((R, D), lambda i: (i, 0)),
        ],
        out_specs=pl.BlockSpec((R, D), lambda i: (i, 0)),
        out_shape=jax.ShapeDtypeStruct((N, D), jnp.float32),
    )(p, selfb)


def kernel(x, edge_index, alpha_sp, alpha_ctx, alpha_lat, w_sp, w_ctx, w_lat,
           W_sp0, W_ctx0, W_lat0, W_self0, W_sp1, W_ctx1, W_lat1, W_self1,
           b0, b1, dp0, dp1, ln_g0, ln_b0):
    src = edge_index[0]
    tgt = edge_index[1]
    w3_0 = jnp.concatenate([W_sp0, W_ctx0, W_lat0], axis=0)
    w3_1 = jnp.concatenate([W_sp1, W_ctx1, W_lat1], axis=0)

    y0, self0 = _tc_transform(x, w3_0, W_self0)

    a3 = jnp.stack([alpha_sp, alpha_ctx, alpha_lat])
    w3g = jnp.stack([w_sp, w_ctx, w_lat])
    eff3 = _tc_eff(a3, w3g)
    effpack = eff3.reshape(3, E // K, K).transpose(1, 0, 2).reshape(-1)

    deg2 = _sc_degree(tgt, effpack)

    dpb0 = jnp.broadcast_to(dp0[:, None], (3, NP))
    dpb1 = jnp.broadcast_to(dp1[:, None], (3, NP))
    d0, d1 = _tc_dpow(deg2, dpb0, dpb1)

    p0 = _sc_msgpass(y0, src, tgt, effpack, d0)[:, :N]
    y1, self1b = _tc_mid(p0, self0, b0.reshape(1, D), ln_g0.reshape(1, D),
                         ln_b0.reshape(1, D), w3_1, W_self1, b1.reshape(1, D))
    p1 = _sc_msgpass(y1, src, tgt, effpack, d1)[:, :N]
    return _tc_final(p1, self1b)


# speedup vs baseline: 15.6077x; 1.1810x over previous
import functools

import jax
import jax.numpy as jnp
from jax import lax
from jax.experimental import pallas as pl
from jax.experimental.pallas import tpu as pltpu
from jax.experimental.pallas import tpu_sc as plsc

N = 10000
E = 320000
D = 128
D3 = 3 * D

NC = 2
NS = 16
NW = NC * NS
EPW = E // NW
K = 80
NCHUNK = EPW // K
NP = 10240
RPT = NP // NS

_SC_MESH = plsc.VectorSubcoreMesh(core_axis_name="c", subcore_axis_name="s")


def _tc_transform_body(x_ref, w3_ref, wself_ref, y_ref, self_ref):
    xb = x_ref[...]
    dn = (((1,), (1,)), ((), ()))
    y_ref[...] = lax.dot_general(xb, w3_ref[...], dn,
                                 preferred_element_type=jnp.float32)
    self_ref[...] = lax.dot_general(xb, wself_ref[...], dn,
                                    preferred_element_type=jnp.float32)


def _tc_transform(x, w3, wself):
    R = 2000
    return pl.pallas_call(
        _tc_transform_body,
        grid=(N // R,),
        in_specs=[
            pl.BlockSpec((R, D), lambda i: (i, 0)),
            pl.BlockSpec((D3, D), lambda i: (0, 0)),
            pl.BlockSpec((D, D), lambda i: (0, 0)),
        ],
        out_specs=[
            pl.BlockSpec((R, D3), lambda i: (i, 0)),
            pl.BlockSpec((R, D), lambda i: (i, 0)),
        ],
        out_shape=[
            jax.ShapeDtypeStruct((N, D3), jnp.float32),
            jax.ShapeDtypeStruct((N, D), jnp.float32),
        ],
    )(x, w3, wself)


def _tc_eff_body(a_ref, w_ref, eff_ref):
    eff_ref[...] = a_ref[...] * w_ref[...]


def _tc_eff(a3, w3):
    RB = 64000
    return pl.pallas_call(
        _tc_eff_body,
        grid=(E // RB,),
        in_specs=[
            pl.BlockSpec((3, RB), lambda i: (0, i)),
            pl.BlockSpec((3, RB), lambda i: (0, i)),
        ],
        out_specs=pl.BlockSpec((3, RB), lambda i: (0, i)),
        out_shape=jax.ShapeDtypeStruct((3, E), jnp.float32),
    )(a3, w3)


DEGB = 5
DEGIT = EPW // (DEGB * K)


def _sc_degree_body(tgt_hbm, effpack_hbm, z1_hbm,
                    out_hbm,
                    tgt_v, eff_v, acc_sp, acc_ctx, acc_lat, sem):
    c = lax.axis_index("c")
    s = lax.axis_index("s")
    wid = s * NC + c
    r0 = s * RPT
    for acc in (acc_sp, acc_ctx, acc_lat):
        pltpu.sync_copy(z1_hbm.at[pl.ds(r0, RPT)], acc.at[pl.ds(r0, RPT)])
    plsc.subcore_barrier()

    accs = (acc_sp, acc_ctx, acc_lat)

    def it(bi, carry):
        e0 = wid * EPW + bi * (DEGB * K)
        p0 = (wid * NCHUNK + bi * DEGB) * (3 * K)
        cps = [pltpu.async_copy(effpack_hbm.at[pl.ds(p0, DEGB * 3 * K)],
                                eff_v, sem)]
        for b in range(DEGB):
            cps.append(pltpu.async_copy(tgt_hbm.at[pl.ds(e0 + b * K, K)],
                                        tgt_v.at[b], sem))
        for cp in cps:
            cp.wait()
        cps = []
        for b in range(DEGB):
            for ch in range(3):
                cps.append(pltpu.async_copy(
                    eff_v.at[pl.ds((b * 3 + ch) * K, K)],
                    accs[ch].at[tgt_v.at[b]], sem, add=True))
        for cp in cps:
            cp.wait()
        return carry

    lax.fori_loop(0, DEGIT, it, 0)
    plsc.subcore_barrier()
    for ch in range(3):
        pltpu.sync_copy(accs[ch].at[pl.ds(r0, RPT)],
                        out_hbm.at[pl.ds(c * (3 * NP) + ch * NP + r0, RPT)])


def _sc_degree(tgt, effpack):
    z1 = jnp.zeros((NP,), jnp.float32)
    f = pl.kernel(
        _sc_degree_body,
        out_type=jax.ShapeDtypeStruct((NC * 3 * NP,), jnp.float32),
        mesh=_SC_MESH,
        scratch_types=[
            pltpu.VMEM((DEGB, K), jnp.int32),
            pltpu.VMEM((DEGB * 3 * K,), jnp.float32),
            pltpu.VMEM_SHARED((NP,), jnp.float32),
            pltpu.VMEM_SHARED((NP,), jnp.float32),
            pltpu.VMEM_SHARED((NP,), jnp.float32),
            pltpu.SemaphoreType.DMA,
        ],
    )
    return f(tgt, effpack, z1).reshape(NC, 3, NP)


def _tc_dpow_body(deg2_ref, dp0_ref, dp1_ref, d0_ref, d1_ref):
    deg = 1.0 + deg2_ref[0] + deg2_ref[1]
    degc = jnp.maximum(deg, 1e-6)
    d0_ref[...] = jnp.power(degc, dp0_ref[...])
    d1_ref[...] = jnp.power(degc, dp1_ref[...])


def _tc_dpow(deg2, dpb0, dpb1):
    R = 2048
    return pl.pallas_call(
        _tc_dpow_body,
        grid=(NP // R,),
        in_specs=[
            pl.BlockSpec((NC, 3, R), lambda i: (0, 0, i)),
            pl.BlockSpec((3, R), lambda i: (0, i)),
            pl.BlockSpec((3, R), lambda i: (0, i)),
        ],
        out_specs=[
            pl.BlockSpec((3, R), lambda i: (0, i)),
            pl.BlockSpec((3, R), lambda i: (0, i)),
        ],
        out_shape=[
            jax.ShapeDtypeStruct((3, NP), jnp.float32),
            jax.ShapeDtypeStruct((3, NP), jnp.float32),
        ],
    )(deg2, dpb0, dpb1)


def _sc_msgpass_body(y_hbm, src_hbm, tgt_hbm, effpack_hbm,
                     dsp_hbm, dctx_hbm, dlat_hbm, zd_hbm,
                     out_hbm,
                     srcA, tgtA, effA, dsA0, dsA1, dsA2, dtA0, dtA1, dtA2,
                     srcB, tgtB, effB, dsB0, dsB1, dsB2, dtB0, dtB1, dtB2,
                     rows_v, csp_v, cctx_v, clat_v, msg_v, acc_sh,
                     semlA, semlB, semdA, semdB, semr):
    c = lax.axis_index("c")
    s = lax.axis_index("s")
    wid = s * NC + c
    r0 = s * RPT
    pltpu.sync_copy(zd_hbm.at[pl.ds(r0, RPT)], acc_sh.at[pl.ds(r0, RPT)])
    plsc.subcore_barrier()

    d_hbms = (dsp_hbm, dctx_hbm, dlat_hbm)
    bufA = (srcA, tgtA, effA, (dsA0, dsA1, dsA2), (dtA0, dtA1, dtA2),
            semlA, semdA)
    bufB = (srcB, tgtB, effB, (dsB0, dsB1, dsB2), (dtB0, dtB1, dtB2),
            semlB, semdB)
    c_refs = (csp_v, cctx_v, clat_v)

    def lin_copies(ci, buf, mk):
        src_v, tgt_v, eff_v, seml = buf[0], buf[1], buf[2], buf[5]
        e0 = wid * EPW + ci * K
        p0 = (wid * NCHUNK + ci) * (3 * K)
        return [mk(src_hbm.at[pl.ds(e0, K)], src_v, seml),
                mk(tgt_hbm.at[pl.ds(e0, K)], tgt_v, seml),
                mk(effpack_hbm.at[pl.ds(p0, 3 * K)], eff_v, seml)]

    def d_copies(buf, mk):
        src_v, tgt_v, ds, dt, semd = buf[0], buf[1], buf[3], buf[4], buf[6]
        cps = []
        for ch in range(3):
            cps.append(mk(d_hbms[ch].at[src_v], ds[ch], semd))
            cps.append(mk(d_hbms[ch].at[tgt_v], dt[ch], semd))
        return cps

    def drain(cps):
        for cp in cps:
            cp.wait()

    def fire_lin(ci, buf):
        lin_copies(ci, buf, pltpu.async_copy)

    def drain_lin(ci, buf):
        drain(lin_copies(ci, buf, pltpu.make_async_copy))

    def fire_d(buf):
        d_copies(buf, pltpu.async_copy)

    def drain_d(buf):
        drain(d_copies(buf, pltpu.make_async_copy))

    def fire_rows(buf):
        pltpu.async_copy(y_hbm.at[buf[0]], rows_v, semr)

    def drain_rows(buf):
        pltpu.make_async_copy(y_hbm.at[buf[0]], rows_v, semr).wait()

    def coeff_part(buf):
        eff_v, ds, dt = buf[2], buf[3], buf[4]
        for g in range(K // 16):
            sl = pl.ds(g * 16, 16)
            for ch in range(3):
                c_refs[ch][sl] = (eff_v[pl.ds(ch * K + g * 16, 16)]
                                  * ds[ch][sl] * dt[ch][sl])

    def combine_part():
        def group(g, carry2):
            j0 = g * 16
            cs = csp_v[pl.ds(j0, 16)]
            cc = cctx_v[pl.ds(j0, 16)]
            cl = clat_v[pl.ds(j0, 16)]
            for jj in range(16):
                c0 = cs[jj]
                c1 = cc[jj]
                c2 = cl[jj]
                j = j0 + jj
                for g8 in range(D // 16):
                    off = g8 * 16
                    v0 = rows_v[j, pl.ds(off, 16)]
                    v1 = rows_v[j, pl.ds(D + off, 16)]
                    v2 = rows_v[j, pl.ds(2 * D + off, 16)]
                    msg_v[j, pl.ds(off, 16)] = c0 * v0 + c1 * v1 + c2 * v2
            return carry2

        lax.fori_loop(0, K // 16, group, 0)

    def step(ci, p, q, last):
        if not last:
            drain_lin(ci + 1, q)
            fire_d(q)
        drain_d(p)
        coeff_part(p)
        drain_rows(p)
        combine_part()
        if not last:
            fire_rows(q)
        pltpu.sync_copy(msg_v, acc_sh.at[p[1]], add=True)
        if not last:
            if isinstance(ci, int):
                if ci + 2 < NCHUNK:
                    fire_lin(ci + 2, p)
            else:
                @pl.when(ci + 2 < NCHUNK)
                def _():
                    fire_lin(ci + 2, p)

    fire_lin(0, bufA)
    drain_lin(0, bufA)
    fire_d(bufA)
    fire_rows(bufA)
    fire_lin(1, bufB)

    def body(i, carry):
        step(2 * i, bufA, bufB, False)
        step(2 * i + 1, bufB, bufA, False)
        return carry

    lax.fori_loop(0, (NCHUNK - 1) // 2, body, 0)
    step(NCHUNK - 1, bufA, bufB, True)

    plsc.subcore_barrier()
    pltpu.sync_copy(acc_sh.at[pl.ds(r0, RPT)], out_hbm.at[c, pl.ds(r0, RPT)])


def _sc_msgpass(y, src, tgt, effpack, d3np):
    zd = jnp.zeros((NP, D), jnp.float32)
    buf = ([pltpu.VMEM((K,), jnp.int32)] * 2
           + [pltpu.VMEM((3 * K,), jnp.float32)]
           + [pltpu.VMEM((K,), jnp.float32)] * 6)
    f = pl.kernel(
        _sc_msgpass_body,
        out_type=jax.ShapeDtypeStruct((NC, NP, D), jnp.float32),
        mesh=_SC_MESH,
        scratch_types=(
            buf + buf
            + [pltpu.VMEM((K, D3), jnp.float32)]
            + [pltpu.VMEM((K,), jnp.float32)] * 3
            + [
                pltpu.VMEM((K, D), jnp.float32),
                pltpu.VMEM_SHARED((NP, D), jnp.float32),
                pltpu.SemaphoreType.DMA,
                pltpu.SemaphoreType.DMA,
                pltpu.SemaphoreType.DMA,
                pltpu.SemaphoreType.DMA,
                pltpu.SemaphoreType.DMA,
            ]
        ),
    )
    return f(y, src, tgt, effpack, d3np[0], d3np[1], d3np[2], zd)


def _tc_mid_body(p_ref, self_ref, b0_ref, g_ref, bln_ref, w3_ref, wself_ref,
                 b1_ref, y1_ref, self1_ref):
    conv = p_ref[0] + p_ref[1] + self_ref[...] + b0_ref[...]
    mu = jnp.mean(conv, axis=-1, keepdims=True)
    var = jnp.mean((conv - mu) ** 2, axis=-1, keepdims=True)
    h = (conv - mu) / jnp.sqrt(var + 1e-5) * g_ref[...] + bln_ref[...]
    h = jnp.maximum(h, 0.0)
    dn = (((1,), (1,)), ((), ()))
    y1_ref[...] = lax.dot_general(h, w3_ref[...], dn,
                                  preferred_element_type=jnp.float32)
    self1_ref[...] = lax.dot_general(h, wself_ref[...], dn,
                                     preferred_element_type=jnp.float32) \
        + b1_ref[...]


def _tc_mid(p, self0, b0, g, bln, w3, wself, b1):
    R = 2000
    return pl.pallas_call(
        _tc_mid_body,
        grid=(N // R,),
        in_specs=[
            pl.BlockSpec((NC, R, D), lambda i: (0, i, 0)),
            pl.BlockSpec((R, D), lambda i: (i, 0)),
            pl.BlockSpec((1, D), lambda i: (0, 0)),
            pl.BlockSpec((1, D), lambda i: (0, 0)),
            pl.BlockSpec((1, D), lambda i: (0, 0)),
            pl.BlockSpec((D3, D), lambda i: (0, 0)),
            pl.BlockSpec((D, D), lambda i: (0, 0)),
            pl.BlockSpec((1, D), lambda i: (0, 0)),
        ],
        out_specs=[
            pl.BlockSpec((R, D3), lambda i: (i, 0)),
            pl.BlockSpec((R, D), lambda i: (i, 0)),
        ],
        out_shape=[
            jax.ShapeDtypeStruct((N, D3), jnp.float32),
            jax.ShapeDtypeStruct((N, D), jnp.float32),
        ],
    )(p, self0, b0, g, bln, w3, wself, b1)


def _tc_final_body(p_ref, selfb_ref, out_ref):
    out_ref[...] = p_ref[0] + p_ref[1] + selfb_ref[...]


def _tc_final(p, selfb):
    R = 2000
    return pl.pallas_call(
        _tc_final_body,
        grid=(N // R,),
        in_specs=[
            pl.BlockSpec((NC, R, D), lambda i: (0, i, 0)),
            pl.BlockSpec((R, D), lambda i: (i, 0)),
        ],
        out_specs=pl.BlockSpec((R, D), lambda i: (i, 0)),
        out_shape=jax.ShapeDtypeStruct((N, D), jnp.float32),
    )(p, selfb)


def kernel(x, edge_index, alpha_sp, alpha_ctx, alpha_lat, w_sp, w_ctx, w_lat,
           W_sp0, W_ctx0, W_lat0, W_self0, W_sp1, W_ctx1, W_lat1, W_self1,
           b0, b1, dp0, dp1, ln_g0, ln_b0):
    src = edge_index[0]
    tgt = edge_index[1]
    w3_0 = jnp.concatenate([W_sp0, W_ctx0, W_lat0], axis=0)
    w3_1 = jnp.concatenate([W_sp1, W_ctx1, W_lat1], axis=0)

    y0, self0 = _tc_transform(x, w3_0, W_self0)

    a3 = jnp.stack([alpha_sp, alpha_ctx, alpha_lat])
    w3g = jnp.stack([w_sp, w_ctx, w_lat])
    eff3 = _tc_eff(a3, w3g)
    effpack = eff3.reshape(3, E // K, K).transpose(1, 0, 2).reshape(-1)

    deg2 = _sc_degree(tgt, effpack)

    dpb0 = jnp.broadcast_to(dp0[:, None], (3, NP))
    dpb1 = jnp.broadcast_to(dp1[:, None], (3, NP))
    d0, d1 = _tc_dpow(deg2, dpb0, dpb1)

    p0 = _sc_msgpass(y0, src, tgt, effpack, d0)[:, :N]
    y1, self1b = _tc_mid(p0, self0, b0.reshape(1, D), ln_g0.reshape(1, D),
                         ln_b0.reshape(1, D), w3_1, W_self1, b1.reshape(1, D))
    p1 = _sc_msgpass(y1, src, tgt, effpack, d1)[:, :N]
    return _tc_final(p1, self1b)
